# Initial kernel scaffold; baseline (speedup 1.0000x reference)
import jax
import jax.numpy as jnp
from jax.experimental import pallas as pl


def kernel(x, edge_index, Wl1, Wr1, b1, Wl2, Wr2, b2, Wl3, Wr3, b3, R1, rb1, R2, rb2):
    raise NotImplementedError("write your pallas kernel here")


# speedup vs baseline: 24.0915x; 24.0915x over previous
import functools

import jax
import jax.numpy as jnp
from jax import lax
from jax.experimental import pallas as pl
from jax.experimental.pallas import tpu as pltpu
from jax.experimental.pallas import tpu_sc as plsc

N = 10000
D = 128

NC = 2
NS = 16
NW = NC * NS
CH = 128
NB = 8
NTAB = 10112
RPT = NTAB // NS
ZR = RPT // 4


def _sc_segment_sum(K):
    mesh = plsc.VectorSubcoreMesh(
        core_axis_name="c", subcore_axis_name="s", num_cores=NC, num_subcores=NS
    )

    @functools.partial(
        pl.kernel,
        out_type=jax.ShapeDtypeStruct((NC, NTAB, 16), jnp.float32),
        mesh=mesh,
        scratch_types=[
            pltpu.VMEM((K, CH), jnp.int32),
            pltpu.VMEM((K, CH), jnp.int32),
            pltpu.VMEM((NB, CH, 16), jnp.float32),
            pltpu.VMEM((ZR, 16), jnp.float32),
            pltpu.VMEM_SHARED((NTAB, 16), jnp.float32),
            pltpu.SemaphoreType.DMA,
            pltpu.SemaphoreType.DMA,
        ],
        compiler_params=pltpu.CompilerParams(use_tc_tiling_on_sc=False),
    )
    def k(tab_h, src_h, dst_h, out_h, idx_s, idx_d, rows, zbuf, acc, sem_g, sem_s):
        c = lax.axis_index("c")
        s = lax.axis_index("s")
        w = s * NC + c
        z = jnp.zeros((16,), jnp.float32)
        for i in range(ZR):
            zbuf[i, :] = z
        base = s * RPT
        for r in range(RPT // ZR):
            pltpu.sync_copy(zbuf, acc.at[pl.ds(base + r * ZR, ZR)])
        pltpu.sync_copy(src_h.at[w], idx_s)
        pltpu.sync_copy(dst_h.at[w], idx_d)
        plsc.subcore_barrier()
        for r in range(K // NB):
            gd = [
                pltpu.async_copy(tab_h.at[idx_s.at[r * NB + b]], rows.at[b], sem_g)
                for b in range(NB)
            ]
            for d_ in gd:
                d_.wait()
            sd = [
                pltpu.async_copy(rows.at[b], acc.at[idx_d.at[r * NB + b]], sem_s, add=True)
                for b in range(NB)
            ]
            for d_ in sd:
                d_.wait()
        plsc.subcore_barrier()
        pltpu.sync_copy(acc.at[pl.ds(base, RPT)], out_h.at[c, pl.ds(base, RPT)])

    return k


def _tab_and_side(u):
    row = lax.broadcasted_iota(jnp.int32, (NTAB, 16), 0)
    lane = lax.broadcasted_iota(jnp.int32, (NTAB, 16), 1)
    live = row < N
    t = jnp.where(live, u[:, :16] + jnp.where(lane == 8, 1.0, 0.0), 0.0)
    return t, u[:, 16:32]


def _tc_pre(xp, w, bb):
    def body(x_ref, w_ref, b_ref, t_ref, s_ref):
        u = jnp.dot(x_ref[...], w_ref[...], preferred_element_type=jnp.float32)
        t, sside = _tab_and_side(u)
        t_ref[...] = t
        s_ref[...] = sside + b_ref[...]

    return pl.pallas_call(
        body,
        out_shape=[
            jax.ShapeDtypeStruct((NTAB, 16), jnp.float32),
            jax.ShapeDtypeStruct((NTAB, 16), jnp.float32),
        ],
    )(xp, w, bb)


def _tc_mid(a, sprev, w, bb):
    def body(a_ref, s_ref, w_ref, b_ref, t_ref, so_ref):
        agg = a_ref[0] + a_ref[1]
        cnt = jnp.maximum(agg[:, 8:9], 1.0)
        h = jnp.maximum(agg[:, 0:8] / cnt + s_ref[:, 0:8], 0.0)
        x1 = h + s_ref[:, 8:16]
        u = jnp.dot(x1, w_ref[...], preferred_element_type=jnp.float32)
        t, sside = _tab_and_side(u)
        t_ref[...] = t
        so_ref[...] = sside + b_ref[...]

    return pl.pallas_call(
        body,
        out_shape=[
            jax.ShapeDtypeStruct((NTAB, 16), jnp.float32),
            jax.ShapeDtypeStruct((NTAB, 16), jnp.float32),
        ],
    )(a, sprev, w, bb)


def _tc_final(a, sprev):
    def body(a_ref, s_ref, o_ref):
        agg = a_ref[0] + a_ref[1]
        cnt = jnp.maximum(agg[:, 8:9], 1.0)
        x3 = agg[:, 0:2] / cnt + s_ref[:, 0:2]
        m = jnp.max(x3, axis=-1, keepdims=True)
        lse = m + jnp.log(jnp.sum(jnp.exp(x3 - m), axis=-1, keepdims=True))
        o_ref[...] = (x3 - lse)[:N]

    return pl.pallas_call(
        body,
        out_shape=jax.ShapeDtypeStruct((N, 2), jnp.float32),
    )(a, sprev)


def kernel(x, edge_index, Wl1, Wr1, b1, Wl2, Wr2, b2, Wl3, Wr3, b3, R1, rb1, R2, rb2):
    E = edge_index.shape[1]
    K = -(-E // (NW * CH))
    K = -(-K // NB) * NB
    epad = NW * K * CH
    npad = epad - E
    fill = (N + jnp.arange(npad, dtype=jnp.int32) % (NTAB - N)) if npad else jnp.zeros((0,), jnp.int32)
    srcs = jnp.concatenate([edge_index[0], fill]).reshape(NW, K, CH)
    dsts = jnp.concatenate([edge_index[1], fill]).reshape(NW, K, CH)

    xp = jnp.pad(x, ((0, NTAB - N), (0, 0)))
    z8 = jnp.zeros((D, 8), jnp.float32)
    w1 = jnp.concatenate([Wl1, z8, Wr1, R1], axis=1)
    bb1 = jnp.concatenate([b1, rb1]).reshape(1, 16)
    z88 = jnp.zeros((8, 8), jnp.float32)
    w2 = jnp.concatenate([Wl2, z88, Wr2, R2], axis=1)
    bb2 = jnp.concatenate([b2, rb2]).reshape(1, 16)
    z86 = jnp.zeros((8, 6), jnp.float32)
    w3 = jnp.concatenate([Wl3, z86, z88, Wr3, z86, z88], axis=1)
    bb3 = jnp.concatenate([b3, jnp.zeros((14,), jnp.float32)]).reshape(1, 16)

    sc = _sc_segment_sum(K)
    t1, s1 = _tc_pre(xp, w1, bb1)
    a1 = sc(t1, srcs, dsts)
    t2, s2 = _tc_mid(a1, s1, w2, bb2)
    a2 = sc(t2, srcs, dsts)
    t3, s3 = _tc_mid(a2, s2, w3, bb3)
    a3 = sc(t3, srcs, dsts)
    return _tc_final(a3, s3)


# speedup vs baseline: 27.2996x; 1.1332x over previous
import functools

import jax
import jax.numpy as jnp
from jax import lax
from jax.experimental import pallas as pl
from jax.experimental.pallas import tpu as pltpu
from jax.experimental.pallas import tpu_sc as plsc

N = 10000
D = 128

NC = 2
NS = 16
NW = NC * NS
CH = 128
NB = 8
NTAB = 10112
RPT = NTAB // NS
ZR = RPT // 4


NCHUNK = 2500
KPW = NCHUNK // NW
NEXTRA = NCHUNK - KPW * NW


def _sc_segment_sum():
    mesh = plsc.VectorSubcoreMesh(
        core_axis_name="c", subcore_axis_name="s", num_cores=NC, num_subcores=NS
    )

    @functools.partial(
        pl.kernel,
        out_type=jax.ShapeDtypeStruct((NC, NTAB, 16), jnp.float32),
        mesh=mesh,
        scratch_types=[
            pltpu.VMEM((KPW + 1, CH), jnp.int32),
            pltpu.VMEM((KPW + 1, CH), jnp.int32),
            pltpu.VMEM((2, NB, CH, 16), jnp.float32),
            pltpu.VMEM((ZR, 16), jnp.float32),
            pltpu.VMEM_SHARED((NTAB, 16), jnp.float32),
            pltpu.SemaphoreType.DMA,
            pltpu.SemaphoreType.DMA,
        ],
        compiler_params=pltpu.CompilerParams(use_tc_tiling_on_sc=False),
    )
    def k(tab_h, eidx_h, out_h, idx_s, idx_d, rows, zbuf, acc, sem_g, sem_s):
        c = lax.axis_index("c")
        s = lax.axis_index("s")
        w = s * NC + c
        z = jnp.zeros((16,), jnp.float32)
        for i in range(ZR):
            zbuf[i, :] = z
        base = s * RPT
        for r in range(RPT // ZR):
            pltpu.sync_copy(zbuf, acc.at[pl.ds(base + r * ZR, ZR)])
        pltpu.sync_copy(eidx_h.at[0, pl.ds(w * KPW, KPW)], idx_s.at[pl.ds(0, KPW)])
        pltpu.sync_copy(eidx_h.at[1, pl.ds(w * KPW, KPW)], idx_d.at[pl.ds(0, KPW)])
        extra = w < NEXTRA

        @pl.when(extra)
        def _():
            pltpu.sync_copy(eidx_h.at[0, NW * KPW + w], idx_s.at[KPW])
            pltpu.sync_copy(eidx_h.at[1, NW * KPW + w], idx_d.at[KPW])

        plsc.subcore_barrier()

        rounds = [range(r0, min(r0 + NB, KPW)) for r0 in range(0, KPW, NB)]

        def fire_gathers(chunks, grp):
            return [
                pltpu.async_copy(tab_h.at[idx_s.at[j]], rows.at[grp, b], sem_g)
                for b, j in enumerate(chunks)
            ]

        gd = fire_gathers(rounds[0], 0)
        for r, chunks in enumerate(rounds):
            grp = r % 2
            for d_ in gd:
                d_.wait()
            if r + 1 < len(rounds):
                gd = fire_gathers(rounds[r + 1], 1 - grp)
            sd = [
                pltpu.async_copy(rows.at[grp, b], acc.at[idx_d.at[j]], sem_s, add=True)
                for b, j in enumerate(chunks)
            ]
            for d_ in sd:
                d_.wait()

        @pl.when(extra)
        def _():
            pltpu.async_copy(tab_h.at[idx_s.at[KPW]], rows.at[0, 0], sem_g).wait()
            pltpu.async_copy(rows.at[0, 0], acc.at[idx_d.at[KPW]], sem_s, add=True).wait()

        plsc.subcore_barrier()
        pltpu.sync_copy(acc.at[pl.ds(base, RPT)], out_h.at[c, pl.ds(base, RPT)])

    return k


def _tab_and_side(u):
    row = lax.broadcasted_iota(jnp.int32, (NTAB, 16), 0)
    lane = lax.broadcasted_iota(jnp.int32, (NTAB, 16), 1)
    live = row < N
    t = jnp.where(live, u[:, :16] + jnp.where(lane == 8, 1.0, 0.0), 0.0)
    return t, u[:, 16:32]


def _tc_pre(xp, w, bb):
    def body(x_ref, w_ref, b_ref, t_ref, s_ref):
        u = jnp.dot(x_ref[...], w_ref[...], preferred_element_type=jnp.float32)
        t, sside = _tab_and_side(u)
        t_ref[...] = t
        s_ref[...] = sside + b_ref[...]

    return pl.pallas_call(
        body,
        out_shape=[
            jax.ShapeDtypeStruct((NTAB, 16), jnp.float32),
            jax.ShapeDtypeStruct((NTAB, 16), jnp.float32),
        ],
    )(xp, w, bb)


def _tc_mid(a, sprev, w, bb):
    def body(a_ref, s_ref, w_ref, b_ref, t_ref, so_ref):
        agg = a_ref[0] + a_ref[1]
        cnt = jnp.maximum(agg[:, 8:9], 1.0)
        h = jnp.maximum(agg[:, 0:8] / cnt + s_ref[:, 0:8], 0.0)
        x1 = h + s_ref[:, 8:16]
        u = jnp.dot(x1, w_ref[...], preferred_element_type=jnp.float32)
        t, sside = _tab_and_side(u)
        t_ref[...] = t
        so_ref[...] = sside + b_ref[...]

    return pl.pallas_call(
        body,
        out_shape=[
            jax.ShapeDtypeStruct((NTAB, 16), jnp.float32),
            jax.ShapeDtypeStruct((NTAB, 16), jnp.float32),
        ],
    )(a, sprev, w, bb)


def _tc_final(a, sprev):
    def body(a_ref, s_ref, o_ref):
        agg = a_ref[0] + a_ref[1]
        cnt = jnp.maximum(agg[:, 8:9], 1.0)
        x3 = agg[:, 0:2] / cnt + s_ref[:, 0:2]
        m = jnp.max(x3, axis=-1, keepdims=True)
        lse = m + jnp.log(jnp.sum(jnp.exp(x3 - m), axis=-1, keepdims=True))
        o_ref[...] = (x3 - lse)[:N]

    return pl.pallas_call(
        body,
        out_shape=jax.ShapeDtypeStruct((N, 2), jnp.float32),
    )(a, sprev)


def kernel(x, edge_index, Wl1, Wr1, b1, Wl2, Wr2, b2, Wl3, Wr3, b3, R1, rb1, R2, rb2):
    eidx = edge_index.reshape(2, NCHUNK, CH)

    xp = jnp.pad(x, ((0, NTAB - N), (0, 0)))
    z8 = jnp.zeros((D, 8), jnp.float32)
    w1 = jnp.concatenate([Wl1, z8, Wr1, R1], axis=1)
    bb1 = jnp.concatenate([b1, rb1]).reshape(1, 16)
    z88 = jnp.zeros((8, 8), jnp.float32)
    w2 = jnp.concatenate([Wl2, z88, Wr2, R2], axis=1)
    bb2 = jnp.concatenate([b2, rb2]).reshape(1, 16)
    z86 = jnp.zeros((8, 6), jnp.float32)
    w3 = jnp.concatenate([Wl3, z86, z88, Wr3, z86, z88], axis=1)
    bb3 = jnp.concatenate([b3, jnp.zeros((14,), jnp.float32)]).reshape(1, 16)

    sc = _sc_segment_sum()
    t1, s1 = _tc_pre(xp, w1, bb1)
    a1 = sc(t1, eidx)
    t2, s2 = _tc_mid(a1, s1, w2, bb2)
    a2 = sc(t2, eidx)
    t3, s3 = _tc_mid(a2, s2, w3, bb3)
    a3 = sc(t3, eidx)
    return _tc_final(a3, s3)


# speedup vs baseline: 38.0519x; 1.3939x over previous
import functools

import jax
import jax.numpy as jnp
from jax import lax
from jax.experimental import pallas as pl
from jax.experimental.pallas import tpu as pltpu
from jax.experimental.pallas import tpu_sc as plsc

N = 10000
D = 128

NC = 2
NS = 16
NW = NC * NS
CH = 128
NB = 8
NTAB = 10112
RPT = NTAB // NS
ZR = RPT // 4


NCHUNK = 2500
KPW = NCHUNK // NW
NEXTRA = NCHUNK - KPW * NW


def _sc_segment_sum():
    mesh = plsc.VectorSubcoreMesh(
        core_axis_name="c", subcore_axis_name="s", num_cores=NC, num_subcores=NS
    )

    @functools.partial(
        pl.kernel,
        out_type=jax.ShapeDtypeStruct((NC, NTAB, 16), jnp.float32),
        mesh=mesh,
        scratch_types=[
            pltpu.VMEM((KPW + 1, CH), jnp.int32),
            pltpu.VMEM((KPW + 1, CH), jnp.int32),
            pltpu.VMEM((2, NB, CH, 16), jnp.float32),
            pltpu.VMEM((ZR, 16), jnp.float32),
            pltpu.VMEM_SHARED((NTAB, 16), jnp.float32),
            pltpu.SemaphoreType.DMA,
            pltpu.SemaphoreType.DMA,
        ],
        compiler_params=pltpu.CompilerParams(use_tc_tiling_on_sc=False),
    )
    def k(tab_h, eidx_h, out_h, idx_s, idx_d, rows, zbuf, acc, sem_g, sem_s):
        c = lax.axis_index("c")
        s = lax.axis_index("s")
        w = s * NC + c
        z = jnp.zeros((16,), jnp.float32)
        for i in range(ZR):
            zbuf[i, :] = z
        base = s * RPT
        for r in range(RPT // ZR):
            pltpu.sync_copy(zbuf, acc.at[pl.ds(base + r * ZR, ZR)])
        pltpu.sync_copy(eidx_h.at[0, pl.ds(w * KPW, KPW)], idx_s.at[pl.ds(0, KPW)])
        pltpu.sync_copy(eidx_h.at[1, pl.ds(w * KPW, KPW)], idx_d.at[pl.ds(0, KPW)])
        extra = w < NEXTRA

        @pl.when(extra)
        def _():
            pltpu.sync_copy(eidx_h.at[0, NW * KPW + w], idx_s.at[KPW])
            pltpu.sync_copy(eidx_h.at[1, NW * KPW + w], idx_d.at[KPW])

        plsc.subcore_barrier()

        rounds = [range(r0, min(r0 + NB, KPW)) for r0 in range(0, KPW, NB)]

        def fire_gathers(chunks, grp):
            return [
                pltpu.async_copy(tab_h.at[idx_s.at[j]], rows.at[grp, b], sem_g)
                for b, j in enumerate(chunks)
            ]

        gd = fire_gathers(rounds[0], 0)
        for r, chunks in enumerate(rounds):
            grp = r % 2
            for d_ in gd:
                d_.wait()
            if r + 1 < len(rounds):
                gd = fire_gathers(rounds[r + 1], 1 - grp)
            sd = [
                pltpu.async_copy(rows.at[grp, b], acc.at[idx_d.at[j]], sem_s, add=True)
                for b, j in enumerate(chunks)
            ]
            for d_ in sd:
                d_.wait()

        @pl.when(extra)
        def _():
            pltpu.async_copy(tab_h.at[idx_s.at[KPW]], rows.at[0, 0], sem_g).wait()
            pltpu.async_copy(rows.at[0, 0], acc.at[idx_d.at[KPW]], sem_s, add=True).wait()

        plsc.subcore_barrier()
        pltpu.sync_copy(acc.at[pl.ds(base, RPT)], out_h.at[c, pl.ds(base, RPT)])

    return k


NR = NTAB // 8
NLIVE = N // 8

import numpy as np


def _blk(b16):
    return jnp.kron(jnp.eye(8, dtype=jnp.float32), b16)


def _np_blk(b16):
    return np.kron(np.eye(8, dtype=np.float32), b16)


_P16 = np.zeros((16, 16), np.float32)
_P16[8, 0:8] = 1.0
_PCNT = _np_blk(_P16)
_R16 = np.zeros((16, 16), np.float32)
_R16[8:16, 0:8] = np.eye(8, dtype=np.float32)
_PRES = _np_blk(_R16)
_S16 = np.zeros((16, 16), np.float32)
_S16[0, 1] = 1.0
_S16[1, 0] = 1.0
_PSWAP = _np_blk(_S16)
_ONEHOT8 = np.tile(np.eye(16, dtype=np.float32)[8], 8).reshape(1, 128)


def _tab_mask(ncols):
    row = lax.broadcasted_iota(jnp.int32, (NR, ncols), 0)
    lane = lax.broadcasted_iota(jnp.int32, (NR, ncols), 1)
    live = (row < NLIVE).astype(jnp.float32)
    oh = (lane % 16 == 8).astype(jnp.float32)
    return live, oh


def _tc_pre(x3, w, bb):
    def body(x_ref, w_ref, b_ref, t_ref, s_ref):
        live, oh = _tab_mask(16)
        for p in range(8):
            u = jnp.dot(
                x_ref[:, p, :], w_ref[...], preferred_element_type=jnp.float32
            )
            sl = pl.ds(16 * p, 16)
            t_ref[:, sl] = (u[:, :16] + oh) * live
            s_ref[:, sl] = u[:, 16:32] + b_ref[...]

    return pl.pallas_call(
        body,
        out_shape=[
            jax.ShapeDtypeStruct((NR, 128), jnp.float32),
            jax.ShapeDtypeStruct((NR, 128), jnp.float32),
        ],
    )(x3, w, bb)


def _tc_mid(a, sprev, wtblk, wsblk, bbt, pcnt, pres):
    def body(a_ref, s_ref, wt_ref, ws_ref, b_ref, pc_ref, pr_ref, t_ref, so_ref):
        agg = a_ref[0] + a_ref[1]
        recip = 1.0 / jnp.maximum(agg, 1.0)
        rb = jnp.dot(recip, pc_ref[...], preferred_element_type=jnp.float32)
        s = s_ref[...]
        h = jnp.maximum(agg * rb + s, 0.0)
        x1 = h + jnp.dot(s, pr_ref[...], preferred_element_type=jnp.float32)
        u = jnp.dot(x1, wt_ref[...], preferred_element_type=jnp.float32)
        live, oh = _tab_mask(128)
        t_ref[...] = (u + oh) * live
        so_ref[...] = jnp.dot(x1, ws_ref[...], preferred_element_type=jnp.float32) + b_ref[...]

    return pl.pallas_call(
        body,
        out_shape=[
            jax.ShapeDtypeStruct((NR, 128), jnp.float32),
            jax.ShapeDtypeStruct((NR, 128), jnp.float32),
        ],
    )(a, sprev, wtblk, wsblk, bbt, pcnt, pres)


def _tc_final(a, sprev, pcnt, pswap):
    def body(a_ref, s_ref, pc_ref, psw_ref, o_ref):
        agg = a_ref[0] + a_ref[1]
        recip = 1.0 / jnp.maximum(agg, 1.0)
        rb = jnp.dot(recip, pc_ref[...], preferred_element_type=jnp.float32)
        x3 = agg * rb + s_ref[...]
        sw = jnp.dot(x3, psw_ref[...], preferred_element_type=jnp.float32)
        mx = jnp.maximum(x3, sw)
        lse = mx + jnp.log(jnp.exp(x3 - mx) + jnp.exp(sw - mx))
        o_ref[...] = x3 - lse

    return pl.pallas_call(
        body,
        out_shape=jax.ShapeDtypeStruct((NR, 128), jnp.float32),
    )(a, sprev, pcnt, pswap)


def kernel(x, edge_index, Wl1, Wr1, b1, Wl2, Wr2, b2, Wl3, Wr3, b3, R1, rb1, R2, rb2):
    eidx = edge_index.reshape(2, NCHUNK, CH)

    xp = jnp.pad(x, ((0, NTAB - N), (0, 0)))
    x3 = xp.reshape(NR, 8, D)
    z8 = jnp.zeros((D, 8), jnp.float32)
    w1 = jnp.concatenate([Wl1, z8, Wr1, R1], axis=1)
    bb1 = jnp.concatenate([b1, rb1]).reshape(1, 16)

    z16 = jnp.zeros((16, 16), jnp.float32)
    wt2 = _blk(z16.at[:8, :8].set(Wl2))
    ws2 = _blk(z16.at[:8, :8].set(Wr2).at[:8, 8:16].set(R2))
    bbt2 = jnp.tile(jnp.concatenate([b2, rb2]), 8).reshape(1, 128)
    wt3 = _blk(z16.at[:8, :2].set(Wl3))
    ws3 = _blk(z16.at[:8, :2].set(Wr3))
    bbt3 = jnp.tile(
        jnp.concatenate([b3, jnp.zeros((14,), jnp.float32)]), 8
    ).reshape(1, 128)

    pcnt = jnp.asarray(_PCNT)
    pres = jnp.asarray(_PRES)
    pswap = jnp.asarray(_PSWAP)

    sc = _sc_segment_sum()
    t1, s1 = _tc_pre(x3, w1, bb1)
    a1 = sc(t1.reshape(NTAB, 16), eidx)
    t2, s2 = _tc_mid(a1.reshape(NC, NR, 128), s1, wt2, ws2, bbt2, pcnt, pres)
    a2 = sc(t2.reshape(NTAB, 16), eidx)
    t3, s3 = _tc_mid(a2.reshape(NC, NR, 128), s2, wt3, ws3, bbt3, pcnt, pres)
    a3 = sc(t3.reshape(NTAB, 16), eidx)
    out = _tc_final(a3.reshape(NC, NR, 128), s3, pcnt, pswap)
    return out.reshape(NTAB, 16)[:N, :2]
